# baseline (speedup 1.0000x reference)
import functools

import jax
import jax.numpy as jnp
from jax import lax
from jax.experimental import pallas as pl
from jax.experimental.pallas import tpu as pltpu
from jax.experimental.pallas import tpu_sc as plsc

_NW = 32


def _make_spmm(V, W):
    Vw = V // _NW
    C = max(2, 4096 // W)
    E = 8 * C
    nchunk = Vw // C
    NS = 3
    assert nchunk >= NS
    ntrip = nchunk // NS
    tail = nchunk % NS
    mesh = plsc.VectorSubcoreMesh(core_axis_name="c", subcore_axis_name="s")

    def body(x_hbm, cols_hbm, vb_hbm, out_hbm, colsv,
             gbuf0, gbuf1, gbuf2, vbuf0, vbuf1, vbuf2, obuf0, obuf1, obuf2,
             sg0, sg1, sg2, sv0, sv1, sv2, so0, so1, so2):
        gbufs = (gbuf0, gbuf1, gbuf2)
        vbufs = (vbuf0, vbuf1, vbuf2)
        obufs = (obuf0, obuf1, obuf2)
        sgs = (sg0, sg1, sg2)
        svs = (sv0, sv1, sv2)
        sos = (so0, so1, so2)
        wid = lax.axis_index("s") * 2 + lax.axis_index("c")
        vbase = wid * Vw
        ebase = vbase * 8
        pltpu.sync_copy(cols_hbm.at[pl.ds(ebase, 8 * Vw)], colsv)

        def issue_loads(g, s):
            pltpu.async_copy(vb_hbm.at[pl.ds(ebase + g * E, E)],
                             vbufs[s], svs[s])
            pltpu.async_copy(x_hbm.at[colsv.at[pl.ds(g * E, E)]],
                             gbufs[s], sgs[s])

        def wait_loads(g, s):
            pltpu.make_async_copy(vb_hbm.at[pl.ds(ebase + g * E, E)],
                                  vbufs[s], svs[s]).wait()
            pltpu.make_async_copy(x_hbm.at[colsv.at[pl.ds(g * E, E)]],
                                  gbufs[s], sgs[s]).wait()

        def compute_store(g, s, not_first):
            @pl.when(not_first)
            def _():
                pltpu.make_async_copy(obufs[s], out_hbm.at[pl.ds(vbase, C)],
                                      sos[s]).wait()
            gbuf, vbuf, obuf = gbufs[s], vbufs[s], obufs[s]
            for r in range(C):
                vv = [vbuf[8 * r + j] for j in range(8)]

                def cc_body(cc, c2, r=r, vv=vv):
                    for u in range(2):
                        col = cc * 32 + u * 16
                        acc = vv[0] * gbuf[8 * r, pl.ds(col, 16)]
                        for j in range(1, 8):
                            acc = acc + vv[j] * gbuf[8 * r + j, pl.ds(col, 16)]
                        obuf[r, pl.ds(col, 16)] = acc
                    return c2

                lax.fori_loop(0, W // 32, cc_body, 0)
            pltpu.async_copy(obuf, out_hbm.at[pl.ds(vbase + g * C, C)],
                             sos[s])

        for s in range(NS):
            issue_loads(s, s)

        def trip(i, carry):
            g0 = NS * i
            for k in range(NS):
                wait_loads(g0 + k, k)
                compute_store(g0 + k, k, g0 + k >= NS)

                @pl.when(g0 + k + NS < nchunk)
                def _(k=k):
                    issue_loads(g0 + k + NS, k)

            return carry

        lax.fori_loop(0, ntrip, trip, 0)
        for k in range(tail):
            g = NS * ntrip + k
            wait_loads(g, k)
            compute_store(g, k, True)
        for s in range(NS):
            pltpu.make_async_copy(obufs[s], out_hbm.at[pl.ds(vbase, C)],
                                  sos[s]).wait()

    scratch = [pltpu.VMEM((8 * Vw,), jnp.int32)]
    scratch += [pltpu.VMEM((E, W), jnp.float32)] * 3
    scratch += [pltpu.VMEM((E, 16), jnp.float32)] * 3
    scratch += [pltpu.VMEM((C, W), jnp.float32)] * 3
    scratch += [pltpu.SemaphoreType.DMA] * 9

    return pl.kernel(
        body,
        mesh=mesh,
        out_type=jax.ShapeDtypeStruct((V, W), jnp.float32),
        scratch_types=scratch,
    )


def _fc(x, w1, b1, w2, b2):
    B = x.shape[0]
    K1 = w1.shape[0]
    H = w1.shape[1]
    N = w2.shape[1]
    NC = 4096
    grid = N // NC

    def body(x_ref, w1_ref, b1_ref, w2_ref, b2_ref, o_ref, h1_ref):
        @pl.when(pl.program_id(0) == 0)
        def _():
            h1_ref[...] = jnp.maximum(
                jnp.dot(x_ref[...], w1_ref[...],
                        preferred_element_type=jnp.float32)
                + b1_ref[...][None, :], 0.0)

        o_ref[...] = (jnp.dot(h1_ref[...], w2_ref[...],
                              preferred_element_type=jnp.float32)
                      + b2_ref[...][None, :])

    return pl.pallas_call(
        body,
        grid=(grid,),
        in_specs=[
            pl.BlockSpec((B, K1), lambda j: (0, 0)),
            pl.BlockSpec((K1, H), lambda j: (0, 0)),
            pl.BlockSpec((H,), lambda j: (0,)),
            pl.BlockSpec((H, NC), lambda j: (0, j)),
            pl.BlockSpec((NC,), lambda j: (j,)),
        ],
        out_specs=pl.BlockSpec((B, NC), lambda j: (0, j)),
        out_shape=jax.ShapeDtypeStruct((B, N), jnp.float32),
        scratch_shapes=[pltpu.VMEM((B, H), jnp.float32)],
    )(x, w1, b1, w2, b2)


def _mm3(t0w, t1w, s2w, w3g, biasg, G, Fin, Fout, with_stats):
    V, Wd = t0w.shape
    GFin = G * Fin
    GFout = G * Fout
    ngb = Wd // GFin
    VB = max(1024, min(V, (1 << 22) // (GFin * 4)))
    grid = (V // VB, ngb)

    def body(*refs):
        if with_stats:
            t0_ref, t1_ref, s2_ref, w_ref, b_ref, y_ref, su_ref, sq_ref = refs
        else:
            t0_ref, t1_ref, s2_ref, w_ref, b_ref, y_ref = refs
        t0b = t0_ref[...]
        acc = (jnp.dot(t0b, w_ref[0], preferred_element_type=jnp.float32)
               + jnp.dot(t1_ref[...], w_ref[1],
                         preferred_element_type=jnp.float32)
               + jnp.dot(s2_ref[...] - t0b, w_ref[2],
                         preferred_element_type=jnp.float32)
               + b_ref[0][None, :])
        y_ref[...] = acc
        if with_stats:
            @pl.when((pl.program_id(0) == 0) & (pl.program_id(1) == 0))
            def _():
                su_ref[...] = jnp.zeros_like(su_ref)
                sq_ref[...] = jnp.zeros_like(sq_ref)

            su_ref[...] += jnp.broadcast_to(
                jnp.sum(acc, axis=0, keepdims=True), (8, GFout))
            sq_ref[...] += jnp.broadcast_to(
                jnp.sum(acc * acc, axis=0, keepdims=True), (8, GFout))

    in_specs = [
        pl.BlockSpec((VB, GFin), lambda i, g: (i, g)),
        pl.BlockSpec((VB, GFin), lambda i, g: (i, g)),
        pl.BlockSpec((VB, GFin), lambda i, g: (i, g)),
        pl.BlockSpec((3, GFin, GFout), lambda i, g: (0, 0, 0)),
        pl.BlockSpec((1, GFout), lambda i, g: (0, 0)),
    ]
    out_shapes = [jax.ShapeDtypeStruct((V, ngb * GFout), jnp.float32)]
    out_specs = [pl.BlockSpec((VB, GFout), lambda i, g: (i, g))]
    if with_stats:
        out_shapes += [jax.ShapeDtypeStruct((8, GFout), jnp.float32)] * 2
        out_specs += [pl.BlockSpec((8, GFout), lambda i, g: (0, 0))] * 2

    res = pl.pallas_call(
        body,
        grid=grid,
        in_specs=in_specs,
        out_specs=out_specs,
        out_shape=out_shapes,
    )(t0w, t1w, s2w, w3g, biasg)
    return res if with_stats else res[0]


def _bn_relu(yw, su_w, sq_w, g_w, b_w, inv_r, expand=1):
    V, Wd = yw.shape
    VBi = max(256, min(V, (1 << 22) // (Wd * 4 * expand)))
    VBo = VBi * expand
    grid = V // VBi

    def body(y_ref, su_ref, sq_ref, g_ref, b_ref, o_ref):
        m = su_ref[0] * inv_r
        var = sq_ref[0] * inv_r - m * m
        scale = g_ref[0] * lax.rsqrt(var + 1e-5)
        shift = b_ref[0] - m * scale
        h = jnp.maximum(y_ref[...] * scale[None, :] + shift[None, :], 0.0)
        if expand > 1:
            h = jnp.broadcast_to(h[:, None, :], (VBi, expand, Wd))
            h = h.reshape(VBo, Wd)
        o_ref[...] = h

    return pl.pallas_call(
        body,
        grid=(grid,),
        in_specs=[
            pl.BlockSpec((VBi, Wd), lambda i: (i, 0)),
            pl.BlockSpec((1, Wd), lambda i: (0, 0)),
            pl.BlockSpec((1, Wd), lambda i: (0, 0)),
            pl.BlockSpec((1, Wd), lambda i: (0, 0)),
            pl.BlockSpec((1, Wd), lambda i: (0, 0)),
        ],
        out_specs=pl.BlockSpec((VBo, Wd), lambda i: (i, 0)),
        out_shape=jax.ShapeDtypeStruct((V * expand, Wd), jnp.float32),
    )(yw, su_w, sq_w, g_w, b_w)


def _cheby(X, V, B, Fin, cols, valsb, valsb2, W, bias, with_stats, G):
    Wd = B * Fin
    Fout = W.shape[1]
    w3 = W.reshape(Fin, 3, Fout).transpose(1, 0, 2)
    eye = jnp.eye(G, dtype=jnp.float32)
    w3g = jnp.stack([jnp.kron(eye, w3[k]) for k in range(3)])
    biasg = jnp.tile(bias, G).reshape(1, G * Fout)
    spmm = _make_spmm(V, Wd)
    t1 = spmm(X, cols, valsb)
    s2 = spmm(t1, cols, valsb2)
    return _mm3(X, t1, s2, w3g, biasg, G, Fin, Fout, with_stats)


def _tile_b(v, B):
    return jnp.tile(v, B).reshape(1, -1)


def kernel(x, fc1_W, fc1_b, fc2_W, fc2_b, cl0_W, cl0_b, g0, b0,
           cl1_W, cl1_b, g1, b1, cl2_W, cl2_b, g2, b2, cl3_W, cl3_b,
           L3_val, L1_val, L3_rows, L3_cols, L1_rows, L1_cols):
    B = x.shape[0]
    V0 = fc2_W.shape[1] // 64
    V3 = 4 * V0
    V1 = 16 * V0

    vb3 = jnp.broadcast_to(L3_val[:, None], (L3_val.shape[0], 16))
    vb3_2 = jnp.broadcast_to(2.0 * L3_val[:, None], (L3_val.shape[0], 16))
    vb1 = jnp.broadcast_to(L1_val[:, None], (L1_val.shape[0], 16))
    vb1_2 = jnp.broadcast_to(2.0 * L1_val[:, None], (L1_val.shape[0], 16))

    h2 = _fc(x, fc1_W, fc1_b, fc2_W, fc2_b)
    h = h2.reshape(B, V0, 64).transpose(1, 0, 2)
    X = jnp.repeat(h.reshape(V0, B * 64), 4, axis=0)

    def bn(yw, su, sq, gg, bb, G, Fout, V, expand=1):
        suT = _tile_b(su[0].reshape(G, Fout).sum(0), B)
        sqT = _tile_b(sq[0].reshape(G, Fout).sum(0), B)
        return _bn_relu(yw, suT, sqT, _tile_b(gg, B), _tile_b(bb, B),
                        1.0 / (V * B), expand=expand)

    y, su, sq = _cheby(X, V3, B, 64, L3_cols, vb3, vb3_2, cl0_W, cl0_b,
                       True, G=2)
    X = bn(y, su, sq, g0, b0, 2, 64, V3)

    y, su, sq = _cheby(X, V3, B, 64, L3_cols, vb3, vb3_2, cl1_W, cl1_b,
                       True, G=4)
    X = bn(y, su, sq, g1, b1, 4, 32, V3, expand=4)

    y, su, sq = _cheby(X, V1, B, 32, L1_cols, vb1, vb1_2, cl2_W, cl2_b,
                       True, G=4)
    X = bn(y, su, sq, g2, b2, 4, 32, V1)

    y = _cheby(X, V1, B, 32, L1_cols, vb1, vb1_2, cl3_W, cl3_b,
               False, G=B)
    return y.reshape(V1, B, 3).transpose(1, 0, 2)


# speedup vs baseline: 1.0207x; 1.0207x over previous
import functools

import jax
import jax.numpy as jnp
from jax import lax
from jax.experimental import pallas as pl
from jax.experimental.pallas import tpu as pltpu
from jax.experimental.pallas import tpu_sc as plsc

_NW = 32


def _make_spmm(V, W):
    Vw = V // _NW
    C = max(2, 4096 // W)
    E = 8 * C
    nchunk = Vw // C
    NS = 2
    assert nchunk >= NS
    ntrip = nchunk // NS
    tail = nchunk % NS
    mesh = plsc.VectorSubcoreMesh(core_axis_name="c", subcore_axis_name="s")

    def body(x_hbm, cols_hbm, vb_hbm, out_hbm, colsv,
             gbuf0, gbuf1, vbuf0, vbuf1, obuf0, obuf1,
             sg0, sg1, sv0, sv1, so0, so1):
        gbufs = (gbuf0, gbuf1)
        vbufs = (vbuf0, vbuf1)
        obufs = (obuf0, obuf1)
        sgs = (sg0, sg1)
        svs = (sv0, sv1)
        sos = (so0, so1)
        wid = lax.axis_index("s") * 2 + lax.axis_index("c")
        vbase = wid * Vw
        ebase = vbase * 8
        pltpu.sync_copy(cols_hbm.at[pl.ds(ebase, 8 * Vw)], colsv)

        def issue_loads(g, s):
            pltpu.async_copy(vb_hbm.at[pl.ds(ebase + g * E, E)],
                             vbufs[s], svs[s])
            pltpu.async_copy(x_hbm.at[colsv.at[pl.ds(g * E, E)]],
                             gbufs[s], sgs[s])

        def wait_loads(g, s):
            pltpu.make_async_copy(vb_hbm.at[pl.ds(ebase + g * E, E)],
                                  vbufs[s], svs[s]).wait()
            pltpu.make_async_copy(x_hbm.at[colsv.at[pl.ds(g * E, E)]],
                                  gbufs[s], sgs[s]).wait()

        def compute_store(g, s, not_first):
            @pl.when(not_first)
            def _():
                pltpu.make_async_copy(obufs[s], out_hbm.at[pl.ds(vbase, C)],
                                      sos[s]).wait()
            gbuf, vbuf, obuf = gbufs[s], vbufs[s], obufs[s]
            for r in range(C):
                vv = [vbuf[8 * r + j] for j in range(8)]

                def cc_body(cc, c2, r=r, vv=vv):
                    for u in range(2):
                        col = cc * 32 + u * 16
                        acc = vv[0] * gbuf[8 * r, pl.ds(col, 16)]
                        for j in range(1, 8):
                            acc = acc + vv[j] * gbuf[8 * r + j, pl.ds(col, 16)]
                        obuf[r, pl.ds(col, 16)] = acc
                    return c2

                lax.fori_loop(0, W // 32, cc_body, 0)
            pltpu.async_copy(obuf, out_hbm.at[pl.ds(vbase + g * C, C)],
                             sos[s])

        for s in range(NS):
            issue_loads(s, s)

        def trip(i, carry):
            g0 = NS * i
            for k in range(NS):
                wait_loads(g0 + k, k)
                compute_store(g0 + k, k, g0 + k >= NS)

                @pl.when(g0 + k + NS < nchunk)
                def _(k=k):
                    issue_loads(g0 + k + NS, k)

            return carry

        lax.fori_loop(0, ntrip, trip, 0)
        for k in range(tail):
            g = NS * ntrip + k
            wait_loads(g, k)
            compute_store(g, k, True)
        for s in range(NS):
            pltpu.make_async_copy(obufs[s], out_hbm.at[pl.ds(vbase, C)],
                                  sos[s]).wait()

    scratch = [pltpu.VMEM((8 * Vw,), jnp.int32)]
    scratch += [pltpu.VMEM((E, W), jnp.float32)] * NS
    scratch += [pltpu.VMEM((E, 16), jnp.float32)] * NS
    scratch += [pltpu.VMEM((C, W), jnp.float32)] * NS
    scratch += [pltpu.SemaphoreType.DMA] * (3 * NS)

    return pl.kernel(
        body,
        mesh=mesh,
        out_type=jax.ShapeDtypeStruct((V, W), jnp.float32),
        scratch_types=scratch,
    )


def _fc(x, w1, b1, w2, b2):
    B = x.shape[0]
    K1 = w1.shape[0]
    H = w1.shape[1]
    N = w2.shape[1]
    NC = 4096
    grid = N // NC

    def body(x_ref, w1_ref, b1_ref, w2_ref, b2_ref, o_ref, h1_ref):
        @pl.when(pl.program_id(0) == 0)
        def _():
            h1_ref[...] = jnp.maximum(
                jnp.dot(x_ref[...], w1_ref[...],
                        preferred_element_type=jnp.float32)
                + b1_ref[...][None, :], 0.0)

        o_ref[...] = (jnp.dot(h1_ref[...], w2_ref[...],
                              preferred_element_type=jnp.float32)
                      + b2_ref[...][None, :])

    return pl.pallas_call(
        body,
        grid=(grid,),
        in_specs=[
            pl.BlockSpec((B, K1), lambda j: (0, 0)),
            pl.BlockSpec((K1, H), lambda j: (0, 0)),
            pl.BlockSpec((H,), lambda j: (0,)),
            pl.BlockSpec((H, NC), lambda j: (0, j)),
            pl.BlockSpec((NC,), lambda j: (j,)),
        ],
        out_specs=pl.BlockSpec((B, NC), lambda j: (0, j)),
        out_shape=jax.ShapeDtypeStruct((B, N), jnp.float32),
        scratch_shapes=[pltpu.VMEM((B, H), jnp.float32)],
    )(x, w1, b1, w2, b2)


def _mm3(t0w, t1w, s2w, w3g, biasg, G, Fin, Fout, with_stats):
    V, Wd = t0w.shape
    GFin = G * Fin
    GFout = G * Fout
    ngb = Wd // GFin
    VB = max(1024, min(V, (1 << 22) // (GFin * 4)))
    grid = (V // VB, ngb)

    def body(*refs):
        if with_stats:
            t0_ref, t1_ref, s2_ref, w_ref, b_ref, y_ref, su_ref, sq_ref = refs
        else:
            t0_ref, t1_ref, s2_ref, w_ref, b_ref, y_ref = refs
        t0b = t0_ref[...]
        acc = (jnp.dot(t0b, w_ref[0], preferred_element_type=jnp.float32)
               + jnp.dot(t1_ref[...], w_ref[1],
                         preferred_element_type=jnp.float32)
               + jnp.dot(s2_ref[...] - t0b, w_ref[2],
                         preferred_element_type=jnp.float32)
               + b_ref[0][None, :])
        y_ref[...] = acc
        if with_stats:
            @pl.when((pl.program_id(0) == 0) & (pl.program_id(1) == 0))
            def _():
                su_ref[...] = jnp.zeros_like(su_ref)
                sq_ref[...] = jnp.zeros_like(sq_ref)

            su_ref[...] += jnp.broadcast_to(
                jnp.sum(acc, axis=0, keepdims=True), (8, GFout))
            sq_ref[...] += jnp.broadcast_to(
                jnp.sum(acc * acc, axis=0, keepdims=True), (8, GFout))

    in_specs = [
        pl.BlockSpec((VB, GFin), lambda i, g: (i, g)),
        pl.BlockSpec((VB, GFin), lambda i, g: (i, g)),
        pl.BlockSpec((VB, GFin), lambda i, g: (i, g)),
        pl.BlockSpec((3, GFin, GFout), lambda i, g: (0, 0, 0)),
        pl.BlockSpec((1, GFout), lambda i, g: (0, 0)),
    ]
    out_shapes = [jax.ShapeDtypeStruct((V, ngb * GFout), jnp.float32)]
    out_specs = [pl.BlockSpec((VB, GFout), lambda i, g: (i, g))]
    if with_stats:
        out_shapes += [jax.ShapeDtypeStruct((8, GFout), jnp.float32)] * 2
        out_specs += [pl.BlockSpec((8, GFout), lambda i, g: (0, 0))] * 2

    res = pl.pallas_call(
        body,
        grid=grid,
        in_specs=in_specs,
        out_specs=out_specs,
        out_shape=out_shapes,
    )(t0w, t1w, s2w, w3g, biasg)
    return res if with_stats else res[0]


def _bn_relu(yw, su_w, sq_w, g_w, b_w, inv_r, expand=1):
    V, Wd = yw.shape
    VBi = max(256, min(V, (1 << 22) // (Wd * 4 * expand)))
    VBo = VBi * expand
    grid = V // VBi

    def body(y_ref, su_ref, sq_ref, g_ref, b_ref, o_ref):
        m = su_ref[0] * inv_r
        var = sq_ref[0] * inv_r - m * m
        scale = g_ref[0] * lax.rsqrt(var + 1e-5)
        shift = b_ref[0] - m * scale
        h = jnp.maximum(y_ref[...] * scale[None, :] + shift[None, :], 0.0)
        if expand > 1:
            h = jnp.broadcast_to(h[:, None, :], (VBi, expand, Wd))
            h = h.reshape(VBo, Wd)
        o_ref[...] = h

    return pl.pallas_call(
        body,
        grid=(grid,),
        in_specs=[
            pl.BlockSpec((VBi, Wd), lambda i: (i, 0)),
            pl.BlockSpec((1, Wd), lambda i: (0, 0)),
            pl.BlockSpec((1, Wd), lambda i: (0, 0)),
            pl.BlockSpec((1, Wd), lambda i: (0, 0)),
            pl.BlockSpec((1, Wd), lambda i: (0, 0)),
        ],
        out_specs=pl.BlockSpec((VBo, Wd), lambda i: (i, 0)),
        out_shape=jax.ShapeDtypeStruct((V * expand, Wd), jnp.float32),
    )(yw, su_w, sq_w, g_w, b_w)


def _cheby(X, V, B, Fin, cols, valsb, valsb2, W, bias, with_stats, G):
    Wd = B * Fin
    Fout = W.shape[1]
    w3 = W.reshape(Fin, 3, Fout).transpose(1, 0, 2)
    eye = jnp.eye(G, dtype=jnp.float32)
    w3g = jnp.stack([jnp.kron(eye, w3[k]) for k in range(3)])
    biasg = jnp.tile(bias, G).reshape(1, G * Fout)
    spmm = _make_spmm(V, Wd)
    t1 = spmm(X, cols, valsb)
    s2 = spmm(t1, cols, valsb2)
    return _mm3(X, t1, s2, w3g, biasg, G, Fin, Fout, with_stats)


def _tile_b(v, B):
    return jnp.tile(v, B).reshape(1, -1)


def kernel(x, fc1_W, fc1_b, fc2_W, fc2_b, cl0_W, cl0_b, g0, b0,
           cl1_W, cl1_b, g1, b1, cl2_W, cl2_b, g2, b2, cl3_W, cl3_b,
           L3_val, L1_val, L3_rows, L3_cols, L1_rows, L1_cols):
    B = x.shape[0]
    V0 = fc2_W.shape[1] // 64
    V3 = 4 * V0
    V1 = 16 * V0

    vb3 = jnp.broadcast_to(L3_val[:, None], (L3_val.shape[0], 16))
    vb3_2 = jnp.broadcast_to(2.0 * L3_val[:, None], (L3_val.shape[0], 16))
    vb1 = jnp.broadcast_to(L1_val[:, None], (L1_val.shape[0], 16))
    vb1_2 = jnp.broadcast_to(2.0 * L1_val[:, None], (L1_val.shape[0], 16))

    h2 = _fc(x, fc1_W, fc1_b, fc2_W, fc2_b)
    h = h2.reshape(B, V0, 64).transpose(1, 0, 2)
    X = jnp.repeat(h.reshape(V0, B * 64), 4, axis=0)

    def bn(yw, su, sq, gg, bb, G, Fout, V, expand=1):
        suT = _tile_b(su[0].reshape(G, Fout).sum(0), B)
        sqT = _tile_b(sq[0].reshape(G, Fout).sum(0), B)
        return _bn_relu(yw, suT, sqT, _tile_b(gg, B), _tile_b(bb, B),
                        1.0 / (V * B), expand=expand)

    y, su, sq = _cheby(X, V3, B, 64, L3_cols, vb3, vb3_2, cl0_W, cl0_b,
                       True, G=2)
    X = bn(y, su, sq, g0, b0, 2, 64, V3)

    y, su, sq = _cheby(X, V3, B, 64, L3_cols, vb3, vb3_2, cl1_W, cl1_b,
                       True, G=4)
    X = bn(y, su, sq, g1, b1, 4, 32, V3, expand=4)

    y, su, sq = _cheby(X, V1, B, 32, L1_cols, vb1, vb1_2, cl2_W, cl2_b,
                       True, G=4)
    X = bn(y, su, sq, g2, b2, 4, 32, V1)

    y = _cheby(X, V1, B, 32, L1_cols, vb1, vb1_2, cl3_W, cl3_b,
               False, G=B)
    return y.reshape(V1, B, 3).transpose(1, 0, 2)


# speedup vs baseline: 1.0380x; 1.0169x over previous
import functools

import jax
import jax.numpy as jnp
from jax import lax
from jax.experimental import pallas as pl
from jax.experimental.pallas import tpu as pltpu
from jax.experimental.pallas import tpu_sc as plsc

_NW = 32


def _make_spmm(V, W):
    Vw = V // _NW
    C = max(2, 4096 // W)
    E = 8 * C
    nchunk = Vw // C
    NS = 2
    assert nchunk >= NS
    ntrip = nchunk // NS
    tail = nchunk % NS
    mesh = plsc.VectorSubcoreMesh(core_axis_name="c", subcore_axis_name="s")

    def body(x_hbm, cols_hbm, vb_hbm, out_hbm, colsv,
             gbuf0, gbuf1, vbuf0, vbuf1, obuf0, obuf1,
             sg0, sg1, sv0, sv1, so0, so1):
        gbufs = (gbuf0, gbuf1)
        vbufs = (vbuf0, vbuf1)
        obufs = (obuf0, obuf1)
        sgs = (sg0, sg1)
        svs = (sv0, sv1)
        sos = (so0, so1)
        wid = lax.axis_index("s") * 2 + lax.axis_index("c")
        vbase = wid * Vw
        ebase = vbase * 8
        pltpu.sync_copy(cols_hbm.at[pl.ds(ebase, 8 * Vw)], colsv)

        def issue_loads(g, s):
            pltpu.async_copy(vb_hbm.at[pl.ds(ebase + g * E, E)],
                             vbufs[s], svs[s])
            pltpu.async_copy(x_hbm.at[colsv.at[pl.ds(g * E, E)]],
                             gbufs[s], sgs[s])

        def wait_loads(g, s):
            pltpu.make_async_copy(vb_hbm.at[pl.ds(ebase + g * E, E)],
                                  vbufs[s], svs[s]).wait()
            pltpu.make_async_copy(x_hbm.at[colsv.at[pl.ds(g * E, E)]],
                                  gbufs[s], sgs[s]).wait()

        def compute_store(g, s, not_first):
            @pl.when(not_first)
            def _():
                pltpu.make_async_copy(obufs[s], out_hbm.at[pl.ds(vbase, C)],
                                      sos[s]).wait()
            gbuf, vbuf, obuf = gbufs[s], vbufs[s], obufs[s]
            for r in range(C):
                vv = [vbuf[8 * r + j] for j in range(8)]

                def cc_body(cc, c2, r=r, vv=vv):
                    col = cc * 16
                    acc = vv[0] * gbuf[8 * r, pl.ds(col, 16)]
                    for j in range(1, 8):
                        acc = acc + vv[j] * gbuf[8 * r + j, pl.ds(col, 16)]
                    obuf[r, pl.ds(col, 16)] = acc
                    return c2

                lax.fori_loop(0, W // 16, cc_body, 0)
            pltpu.async_copy(obuf, out_hbm.at[pl.ds(vbase + g * C, C)],
                             sos[s])

        for s in range(NS):
            issue_loads(s, s)

        def trip(i, carry):
            g0 = NS * i
            for k in range(NS):
                wait_loads(g0 + k, k)
                compute_store(g0 + k, k, g0 + k >= NS)

                @pl.when(g0 + k + NS < nchunk)
                def _(k=k):
                    issue_loads(g0 + k + NS, k)

            return carry

        lax.fori_loop(0, ntrip, trip, 0)
        for k in range(tail):
            g = NS * ntrip + k
            wait_loads(g, k)
            compute_store(g, k, True)
        for s in range(NS):
            pltpu.make_async_copy(obufs[s], out_hbm.at[pl.ds(vbase, C)],
                                  sos[s]).wait()

    scratch = [pltpu.VMEM((8 * Vw,), jnp.int32)]
    scratch += [pltpu.VMEM((E, W), jnp.float32)] * NS
    scratch += [pltpu.VMEM((E, 16), jnp.float32)] * NS
    scratch += [pltpu.VMEM((C, W), jnp.float32)] * NS
    scratch += [pltpu.SemaphoreType.DMA] * (3 * NS)

    return pl.kernel(
        body,
        mesh=mesh,
        out_type=jax.ShapeDtypeStruct((V, W), jnp.float32),
        scratch_types=scratch,
    )


def _fc(x, w1, b1, w2, b2):
    B = x.shape[0]
    K1 = w1.shape[0]
    H = w1.shape[1]
    N = w2.shape[1]
    NC = 4096
    grid = N // NC

    def body(x_ref, w1_ref, b1_ref, w2_ref, b2_ref, o_ref, h1_ref):
        @pl.when(pl.program_id(0) == 0)
        def _():
            h1_ref[...] = jnp.maximum(
                jnp.dot(x_ref[...], w1_ref[...],
                        preferred_element_type=jnp.float32)
                + b1_ref[...][None, :], 0.0)

        o_ref[...] = (jnp.dot(h1_ref[...], w2_ref[...],
                              preferred_element_type=jnp.float32)
                      + b2_ref[...][None, :])

    return pl.pallas_call(
        body,
        grid=(grid,),
        in_specs=[
            pl.BlockSpec((B, K1), lambda j: (0, 0)),
            pl.BlockSpec((K1, H), lambda j: (0, 0)),
            pl.BlockSpec((H,), lambda j: (0,)),
            pl.BlockSpec((H, NC), lambda j: (0, j)),
            pl.BlockSpec((NC,), lambda j: (j,)),
        ],
        out_specs=pl.BlockSpec((B, NC), lambda j: (0, j)),
        out_shape=jax.ShapeDtypeStruct((B, N), jnp.float32),
        scratch_shapes=[pltpu.VMEM((B, H), jnp.float32)],
    )(x, w1, b1, w2, b2)


def _mm3(t0w, t1w, s2w, w3g, biasg, G, Fin, Fout, with_stats):
    V, Wd = t0w.shape
    GFin = G * Fin
    GFout = G * Fout
    ngb = Wd // GFin
    VB = max(1024, min(V, (1 << 22) // (GFin * 4)))
    grid = (V // VB, ngb)

    def body(*refs):
        if with_stats:
            t0_ref, t1_ref, s2_ref, w_ref, b_ref, y_ref, su_ref, sq_ref = refs
        else:
            t0_ref, t1_ref, s2_ref, w_ref, b_ref, y_ref = refs
        t0b = t0_ref[...]
        acc = (jnp.dot(t0b, w_ref[0], preferred_element_type=jnp.float32)
               + jnp.dot(t1_ref[...], w_ref[1],
                         preferred_element_type=jnp.float32)
               + jnp.dot(s2_ref[...] - t0b, w_ref[2],
                         preferred_element_type=jnp.float32)
               + b_ref[0][None, :])
        y_ref[...] = acc
        if with_stats:
            @pl.when((pl.program_id(0) == 0) & (pl.program_id(1) == 0))
            def _():
                su_ref[...] = jnp.zeros_like(su_ref)
                sq_ref[...] = jnp.zeros_like(sq_ref)

            su_ref[...] += jnp.broadcast_to(
                jnp.sum(acc, axis=0, keepdims=True), (8, GFout))
            sq_ref[...] += jnp.broadcast_to(
                jnp.sum(acc * acc, axis=0, keepdims=True), (8, GFout))

    in_specs = [
        pl.BlockSpec((VB, GFin), lambda i, g: (i, g)),
        pl.BlockSpec((VB, GFin), lambda i, g: (i, g)),
        pl.BlockSpec((VB, GFin), lambda i, g: (i, g)),
        pl.BlockSpec((3, GFin, GFout), lambda i, g: (0, 0, 0)),
        pl.BlockSpec((1, GFout), lambda i, g: (0, 0)),
    ]
    out_shapes = [jax.ShapeDtypeStruct((V, ngb * GFout), jnp.float32)]
    out_specs = [pl.BlockSpec((VB, GFout), lambda i, g: (i, g))]
    if with_stats:
        out_shapes += [jax.ShapeDtypeStruct((8, GFout), jnp.float32)] * 2
        out_specs += [pl.BlockSpec((8, GFout), lambda i, g: (0, 0))] * 2

    res = pl.pallas_call(
        body,
        grid=grid,
        in_specs=in_specs,
        out_specs=out_specs,
        out_shape=out_shapes,
    )(t0w, t1w, s2w, w3g, biasg)
    return res if with_stats else res[0]


def _bn_relu(yw, su_w, sq_w, g_w, b_w, inv_r, expand=1):
    V, Wd = yw.shape
    VBi = max(256, min(V, (1 << 22) // (Wd * 4 * expand)))
    VBo = VBi * expand
    grid = V // VBi

    def body(y_ref, su_ref, sq_ref, g_ref, b_ref, o_ref):
        m = su_ref[0] * inv_r
        var = sq_ref[0] * inv_r - m * m
        scale = g_ref[0] * lax.rsqrt(var + 1e-5)
        shift = b_ref[0] - m * scale
        h = jnp.maximum(y_ref[...] * scale[None, :] + shift[None, :], 0.0)
        if expand > 1:
            h = jnp.broadcast_to(h[:, None, :], (VBi, expand, Wd))
            h = h.reshape(VBo, Wd)
        o_ref[...] = h

    return pl.pallas_call(
        body,
        grid=(grid,),
        in_specs=[
            pl.BlockSpec((VBi, Wd), lambda i: (i, 0)),
            pl.BlockSpec((1, Wd), lambda i: (0, 0)),
            pl.BlockSpec((1, Wd), lambda i: (0, 0)),
            pl.BlockSpec((1, Wd), lambda i: (0, 0)),
            pl.BlockSpec((1, Wd), lambda i: (0, 0)),
        ],
        out_specs=pl.BlockSpec((VBo, Wd), lambda i: (i, 0)),
        out_shape=jax.ShapeDtypeStruct((V * expand, Wd), jnp.float32),
    )(yw, su_w, sq_w, g_w, b_w)


def _cheby(X, V, B, Fin, cols, valsb, valsb2, W, bias, with_stats, G):
    Wd = B * Fin
    Fout = W.shape[1]
    w3 = W.reshape(Fin, 3, Fout).transpose(1, 0, 2)
    eye = jnp.eye(G, dtype=jnp.float32)
    w3g = jnp.stack([jnp.kron(eye, w3[k]) for k in range(3)])
    biasg = jnp.tile(bias, G).reshape(1, G * Fout)
    spmm = _make_spmm(V, Wd)
    t1 = spmm(X, cols, valsb)
    s2 = spmm(t1, cols, valsb2)
    return _mm3(X, t1, s2, w3g, biasg, G, Fin, Fout, with_stats)


def _tile_b(v, B):
    return jnp.tile(v, B).reshape(1, -1)


def kernel(x, fc1_W, fc1_b, fc2_W, fc2_b, cl0_W, cl0_b, g0, b0,
           cl1_W, cl1_b, g1, b1, cl2_W, cl2_b, g2, b2, cl3_W, cl3_b,
           L3_val, L1_val, L3_rows, L3_cols, L1_rows, L1_cols):
    B = x.shape[0]
    V0 = fc2_W.shape[1] // 64
    V3 = 4 * V0
    V1 = 16 * V0

    vb3 = jnp.broadcast_to(L3_val[:, None], (L3_val.shape[0], 16))
    vb3_2 = jnp.broadcast_to(2.0 * L3_val[:, None], (L3_val.shape[0], 16))
    vb1 = jnp.broadcast_to(L1_val[:, None], (L1_val.shape[0], 16))
    vb1_2 = jnp.broadcast_to(2.0 * L1_val[:, None], (L1_val.shape[0], 16))

    h2 = _fc(x, fc1_W, fc1_b, fc2_W, fc2_b)
    h = h2.reshape(B, V0, 64).transpose(1, 0, 2)
    X = jnp.repeat(h.reshape(V0, B * 64), 4, axis=0)

    def bn(yw, su, sq, gg, bb, G, Fout, V, expand=1):
        suT = _tile_b(su[0].reshape(G, Fout).sum(0), B)
        sqT = _tile_b(sq[0].reshape(G, Fout).sum(0), B)
        return _bn_relu(yw, suT, sqT, _tile_b(gg, B), _tile_b(bb, B),
                        1.0 / (V * B), expand=expand)

    y, su, sq = _cheby(X, V3, B, 64, L3_cols, vb3, vb3_2, cl0_W, cl0_b,
                       True, G=2)
    X = bn(y, su, sq, g0, b0, 2, 64, V3)

    y, su, sq = _cheby(X, V3, B, 64, L3_cols, vb3, vb3_2, cl1_W, cl1_b,
                       True, G=4)
    X = bn(y, su, sq, g1, b1, 4, 32, V3, expand=4)

    y, su, sq = _cheby(X, V1, B, 32, L1_cols, vb1, vb1_2, cl2_W, cl2_b,
                       True, G=4)
    X = bn(y, su, sq, g2, b2, 4, 32, V1)

    y = _cheby(X, V1, B, 32, L1_cols, vb1, vb1_2, cl3_W, cl3_b,
               False, G=B)
    return y.reshape(V1, B, 3).transpose(1, 0, 2)


# speedup vs baseline: 1.0381x; 1.0001x over previous
import jax
import jax.numpy as jnp
from jax import lax
from jax.experimental import pallas as pl
from jax.experimental.pallas import tpu as pltpu
from jax.experimental.pallas import tpu_sc as plsc

_NW = 32


def _make_spmm(V, W):
    Vw = V // _NW
    C = max(2, 4096 // W)
    E = 8 * C
    nchunk = Vw // C
    NS = 2
    assert nchunk >= NS
    ntrip = nchunk // NS
    tail = nchunk % NS
    mesh = plsc.VectorSubcoreMesh(core_axis_name="c", subcore_axis_name="s")

    def body(x_hbm, cols_hbm, vb_hbm, out_hbm, colsv,
             gbuf0, gbuf1, vbuf0, vbuf1, obuf0, obuf1,
             sg0, sg1, sv0, sv1, so0, so1):
        gbufs = (gbuf0, gbuf1)
        vbufs = (vbuf0, vbuf1)
        obufs = (obuf0, obuf1)
        sgs = (sg0, sg1)
        svs = (sv0, sv1)
        sos = (so0, so1)
        wid = lax.axis_index("s") * 2 + lax.axis_index("c")
        vbase = wid * Vw
        ebase = vbase * 8
        pltpu.sync_copy(cols_hbm.at[pl.ds(ebase, 8 * Vw)], colsv)

        def issue_loads(g, s):
            pltpu.async_copy(vb_hbm.at[pl.ds(ebase + g * E, E)],
                             vbufs[s], svs[s])
            pltpu.async_copy(x_hbm.at[colsv.at[pl.ds(g * E, E)]],
                             gbufs[s], sgs[s])

        def wait_loads(g, s):
            pltpu.make_async_copy(vb_hbm.at[pl.ds(ebase + g * E, E)],
                                  vbufs[s], svs[s]).wait()
            pltpu.make_async_copy(x_hbm.at[colsv.at[pl.ds(g * E, E)]],
                                  gbufs[s], sgs[s]).wait()

        def compute_store(g, s, not_first):
            @pl.when(not_first)
            def _():
                pltpu.make_async_copy(obufs[s], out_hbm.at[pl.ds(vbase, C)],
                                      sos[s]).wait()
            gbuf, vbuf, obuf = gbufs[s], vbufs[s], obufs[s]
            for r in range(C):
                vv = [vbuf[8 * r + j] for j in range(8)]

                def cc_body(cc, c2, r=r, vv=vv):
                    col = cc * 16
                    acc = vv[0] * gbuf[8 * r, pl.ds(col, 16)]
                    for j in range(1, 8):
                        acc = acc + vv[j] * gbuf[8 * r + j, pl.ds(col, 16)]
                    obuf[r, pl.ds(col, 16)] = acc
                    return c2

                lax.fori_loop(0, W // 16, cc_body, 0)
            pltpu.async_copy(obuf, out_hbm.at[pl.ds(vbase + g * C, C)],
                             sos[s])

        for s in range(NS):
            issue_loads(s, s)

        def trip(i, carry):
            g0 = NS * i
            for k in range(NS):
                wait_loads(g0 + k, k)
                compute_store(g0 + k, k, g0 + k >= NS)

                @pl.when(g0 + k + NS < nchunk)
                def _(k=k):
                    issue_loads(g0 + k + NS, k)

            return carry

        lax.fori_loop(0, ntrip, trip, 0)
        for k in range(tail):
            g = NS * ntrip + k
            wait_loads(g, k)
            compute_store(g, k, True)
        for s in range(NS):
            pltpu.make_async_copy(obufs[s], out_hbm.at[pl.ds(vbase, C)],
                                  sos[s]).wait()

    scratch = [pltpu.VMEM((8 * Vw,), jnp.int32)]
    scratch += [pltpu.VMEM((E, W), jnp.float32)] * NS
    scratch += [pltpu.VMEM((E, 16), jnp.float32)] * NS
    scratch += [pltpu.VMEM((C, W), jnp.float32)] * NS
    scratch += [pltpu.SemaphoreType.DMA] * (3 * NS)

    return pl.kernel(
        body,
        mesh=mesh,
        out_type=jax.ShapeDtypeStruct((V, W), jnp.float32),
        scratch_types=scratch,
    )


def _fc(x, w1, b1, w2, b2):
    B = x.shape[0]
    K1 = w1.shape[0]
    H = w1.shape[1]
    N = w2.shape[1]
    NC = 4096
    grid = N // NC

    def body(x_ref, w1_ref, b1_ref, w2_ref, b2_ref, o_ref, h1_ref):
        @pl.when(pl.program_id(0) == 0)
        def _():
            h1_ref[...] = jnp.maximum(
                jnp.dot(x_ref[...], w1_ref[...],
                        preferred_element_type=jnp.float32)
                + b1_ref[...][None, :], 0.0)

        o_ref[...] = (jnp.dot(h1_ref[...], w2_ref[...],
                              preferred_element_type=jnp.float32)
                      + b2_ref[...][None, :])

    return pl.pallas_call(
        body,
        grid=(grid,),
        in_specs=[
            pl.BlockSpec((B, K1), lambda j: (0, 0)),
            pl.BlockSpec((K1, H), lambda j: (0, 0)),
            pl.BlockSpec((H,), lambda j: (0,)),
            pl.BlockSpec((H, NC), lambda j: (0, j)),
            pl.BlockSpec((NC,), lambda j: (j,)),
        ],
        out_specs=pl.BlockSpec((B, NC), lambda j: (0, j)),
        out_shape=jax.ShapeDtypeStruct((B, N), jnp.float32),
        scratch_shapes=[pltpu.VMEM((B, H), jnp.float32)],
    )(x, w1, b1, w2, b2)


def _mm3(t0w, t1w, s2w, w3g, biasg, G, Fin, Fout, with_stats):
    V, Wd = t0w.shape
    GFin = G * Fin
    GFout = G * Fout
    ngb = Wd // GFin
    VB = max(1024, min(V, (1 << 22) // (GFin * 4)))
    grid = (V // VB, ngb)

    def body(*refs):
        if with_stats:
            t0_ref, t1_ref, s2_ref, w_ref, b_ref, y_ref, su_ref, sq_ref = refs
        else:
            t0_ref, t1_ref, s2_ref, w_ref, b_ref, y_ref = refs
        t0b = t0_ref[...]
        acc = (jnp.dot(t0b, w_ref[0], preferred_element_type=jnp.float32)
               + jnp.dot(t1_ref[...], w_ref[1],
                         preferred_element_type=jnp.float32)
               + jnp.dot(s2_ref[...] - t0b, w_ref[2],
                         preferred_element_type=jnp.float32)
               + b_ref[0][None, :])
        y_ref[...] = acc
        if with_stats:
            @pl.when((pl.program_id(0) == 0) & (pl.program_id(1) == 0))
            def _():
                su_ref[...] = jnp.zeros_like(su_ref)
                sq_ref[...] = jnp.zeros_like(sq_ref)

            su_ref[...] += jnp.broadcast_to(
                jnp.sum(acc, axis=0, keepdims=True), (8, GFout))
            sq_ref[...] += jnp.broadcast_to(
                jnp.sum(acc * acc, axis=0, keepdims=True), (8, GFout))

    in_specs = [
        pl.BlockSpec((VB, GFin), lambda i, g: (i, g)),
        pl.BlockSpec((VB, GFin), lambda i, g: (i, g)),
        pl.BlockSpec((VB, GFin), lambda i, g: (i, g)),
        pl.BlockSpec((3, GFin, GFout), lambda i, g: (0, 0, 0)),
        pl.BlockSpec((1, GFout), lambda i, g: (0, 0)),
    ]
    out_shapes = [jax.ShapeDtypeStruct((V, ngb * GFout), jnp.float32)]
    out_specs = [pl.BlockSpec((VB, GFout), lambda i, g: (i, g))]
    if with_stats:
        out_shapes += [jax.ShapeDtypeStruct((8, GFout), jnp.float32)] * 2
        out_specs += [pl.BlockSpec((8, GFout), lambda i, g: (0, 0))] * 2

    res = pl.pallas_call(
        body,
        grid=grid,
        in_specs=in_specs,
        out_specs=out_specs,
        out_shape=out_shapes,
    )(t0w, t1w, s2w, w3g, biasg)
    return res if with_stats else res[0]


def _bn_relu(yw, su_w, sq_w, g_w, b_w, inv_r, expand=1):
    V, Wd = yw.shape
    VBi = max(256, min(V, (1 << 22) // (Wd * 4 * expand)))
    VBo = VBi * expand
    grid = V // VBi

    def body(y_ref, su_ref, sq_ref, g_ref, b_ref, o_ref):
        m = su_ref[0] * inv_r
        var = sq_ref[0] * inv_r - m * m
        scale = g_ref[0] * lax.rsqrt(var + 1e-5)
        shift = b_ref[0] - m * scale
        h = jnp.maximum(y_ref[...] * scale[None, :] + shift[None, :], 0.0)
        if expand > 1:
            h = jnp.broadcast_to(h[:, None, :], (VBi, expand, Wd))
            h = h.reshape(VBo, Wd)
        o_ref[...] = h

    return pl.pallas_call(
        body,
        grid=(grid,),
        in_specs=[
            pl.BlockSpec((VBi, Wd), lambda i: (i, 0)),
            pl.BlockSpec((1, Wd), lambda i: (0, 0)),
            pl.BlockSpec((1, Wd), lambda i: (0, 0)),
            pl.BlockSpec((1, Wd), lambda i: (0, 0)),
            pl.BlockSpec((1, Wd), lambda i: (0, 0)),
        ],
        out_specs=pl.BlockSpec((VBo, Wd), lambda i: (i, 0)),
        out_shape=jax.ShapeDtypeStruct((V * expand, Wd), jnp.float32),
    )(yw, su_w, sq_w, g_w, b_w)


def _cheby(X, V, B, Fin, cols, valsb, valsb2, W, bias, with_stats, G):
    Wd = B * Fin
    Fout = W.shape[1]
    w3 = W.reshape(Fin, 3, Fout).transpose(1, 0, 2)
    eye = jnp.eye(G, dtype=jnp.float32)
    w3g = jnp.stack([jnp.kron(eye, w3[k]) for k in range(3)])
    biasg = jnp.tile(bias, G).reshape(1, G * Fout)
    spmm = _make_spmm(V, Wd)
    t1 = spmm(X, cols, valsb)
    s2 = spmm(t1, cols, valsb2)
    return _mm3(X, t1, s2, w3g, biasg, G, Fin, Fout, with_stats)


def _tile_b(v, B):
    return jnp.tile(v, B).reshape(1, -1)


def kernel(x, fc1_W, fc1_b, fc2_W, fc2_b, cl0_W, cl0_b, g0, b0,
           cl1_W, cl1_b, g1, b1, cl2_W, cl2_b, g2, b2, cl3_W, cl3_b,
           L3_val, L1_val, L3_rows, L3_cols, L1_rows, L1_cols):
    B = x.shape[0]
    V0 = fc2_W.shape[1] // 64
    V3 = 4 * V0
    V1 = 16 * V0

    vb3 = jnp.broadcast_to(L3_val[:, None], (L3_val.shape[0], 16))
    vb3_2 = jnp.broadcast_to(2.0 * L3_val[:, None], (L3_val.shape[0], 16))
    vb1 = jnp.broadcast_to(L1_val[:, None], (L1_val.shape[0], 16))
    vb1_2 = jnp.broadcast_to(2.0 * L1_val[:, None], (L1_val.shape[0], 16))

    h2 = _fc(x, fc1_W, fc1_b, fc2_W, fc2_b)
    h = h2.reshape(B, V0, 64).transpose(1, 0, 2)
    X = jnp.repeat(h.reshape(V0, B * 64), 4, axis=0)

    def bn(yw, su, sq, gg, bb, G, Fout, V, expand=1):
        suT = _tile_b(su[0].reshape(G, Fout).sum(0), B)
        sqT = _tile_b(sq[0].reshape(G, Fout).sum(0), B)
        return _bn_relu(yw, suT, sqT, _tile_b(gg, B), _tile_b(bb, B),
                        1.0 / (V * B), expand=expand)

    y, su, sq = _cheby(X, V3, B, 64, L3_cols, vb3, vb3_2, cl0_W, cl0_b,
                       True, G=2)
    X = bn(y, su, sq, g0, b0, 2, 64, V3)

    y, su, sq = _cheby(X, V3, B, 64, L3_cols, vb3, vb3_2, cl1_W, cl1_b,
                       True, G=4)
    X = bn(y, su, sq, g1, b1, 4, 32, V3, expand=4)

    y, su, sq = _cheby(X, V1, B, 32, L1_cols, vb1, vb1_2, cl2_W, cl2_b,
                       True, G=4)
    X = bn(y, su, sq, g2, b2, 4, 32, V1)

    y = _cheby(X, V1, B, 32, L1_cols, vb1, vb1_2, cl3_W, cl3_b,
               False, G=B)
    return y.reshape(V1, B, 3).transpose(1, 0, 2)
